# Initial kernel scaffold; baseline (speedup 1.0000x reference)
import jax
import jax.numpy as jnp
from jax import lax
from jax.experimental import pallas as pl
from jax.experimental.pallas import tpu as pltpu
from jax.experimental.pallas import tpu_sc as plsc

VOCAB = 1000000
DIM = 64
B = 16384
C = 20
K = 50

NC = 2
NS = 16
NW = NC * NS
EPW = B // NW
CHUNK = 8
NCHUNK = EPW // CHUNK


def _sc_body(cw_hbm, ctx_hbm, neg_hbm, ctab_hbm, xtab_hbm,
             pos_hbm, nsc_hbm,
             cidx_v, xidx_v, nidx_v, crow_v, xrow_v, nrow_v,
             pos_v, neg_v, sem):
    wid = lax.axis_index("s") * NC + lax.axis_index("c")

    def chunk_body(t, carry):
        base = wid * EPW + t * CHUNK
        pltpu.sync_copy(cw_hbm.at[pl.ds(base, CHUNK)], cidx_v)
        pltpu.sync_copy(ctx_hbm.at[pl.ds(base * C, CHUNK * C)], xidx_v)
        pltpu.sync_copy(neg_hbm.at[pl.ds(base * K, CHUNK * K)], nidx_v)
        d1 = pltpu.async_copy(ctab_hbm.at[cidx_v], crow_v, sem)
        d2 = pltpu.async_copy(xtab_hbm.at[xidx_v], xrow_v, sem)
        d3 = pltpu.async_copy(xtab_hbm.at[nidx_v], nrow_v, sem)
        d1.wait()
        d2.wait()
        d3.wait()

        def elem_body(e, ecarry):
            c = [crow_v[e, pl.ds(16 * j, 16)] for j in range(4)]
            cn = [-cj for cj in c]
            for r in range(C):
                row = e * C + r
                t0 = xrow_v[row, pl.ds(0, 16)] * c[0]
                t1 = xrow_v[row, pl.ds(16, 16)] * c[1]
                t2 = xrow_v[row, pl.ds(32, 16)] * c[2]
                t3 = xrow_v[row, pl.ds(48, 16)] * c[3]
                pos_v[e, r] = jnp.sum((t0 + t1) + (t2 + t3))
            for r in range(K):
                row = e * K + r
                t0 = nrow_v[row, pl.ds(0, 16)] * cn[0]
                t1 = nrow_v[row, pl.ds(16, 16)] * cn[1]
                t2 = nrow_v[row, pl.ds(32, 16)] * cn[2]
                t3 = nrow_v[row, pl.ds(48, 16)] * cn[3]
                neg_v[e, r] = jnp.sum((t0 + t1) + (t2 + t3))
            return ecarry

        lax.fori_loop(0, CHUNK, elem_body, 0)
        pltpu.sync_copy(pos_v, pos_hbm.at[pl.ds(base, CHUNK)])
        pltpu.sync_copy(neg_v, nsc_hbm.at[pl.ds(base, CHUNK)])
        return carry

    lax.fori_loop(0, NCHUNK, chunk_body, 0)


_sc_kernel = pl.kernel(
    _sc_body,
    out_type=(
        jax.ShapeDtypeStruct((B, C), jnp.float32),
        jax.ShapeDtypeStruct((B, K), jnp.float32),
    ),
    mesh=plsc.VectorSubcoreMesh(core_axis_name="c", subcore_axis_name="s"),
    scratch_types=[
        pltpu.VMEM((CHUNK,), jnp.int32),
        pltpu.VMEM((CHUNK * C,), jnp.int32),
        pltpu.VMEM((CHUNK * K,), jnp.int32),
        pltpu.VMEM((CHUNK, DIM), jnp.float32),
        pltpu.VMEM((CHUNK * C, DIM), jnp.float32),
        pltpu.VMEM((CHUNK * K, DIM), jnp.float32),
        pltpu.VMEM((CHUNK, C), jnp.float32),
        pltpu.VMEM((CHUNK, K), jnp.float32),
        pltpu.SemaphoreType.DMA,
    ],
)


@jax.jit
def kernel(center_word, context_words, negative_words, centerword_table,
           contextword_table):
    cw = center_word.astype(jnp.int32)
    ctx = context_words.astype(jnp.int32).reshape(B * C)
    neg = negative_words.astype(jnp.int32).reshape(B * K)
    return _sc_kernel(cw, ctx, neg, centerword_table, contextword_table)


# speedup vs baseline: 2.8543x; 2.8543x over previous
import jax
import jax.numpy as jnp
from jax import lax
from jax.experimental import pallas as pl
from jax.experimental.pallas import tpu as pltpu
from jax.experimental.pallas import tpu_sc as plsc

VOCAB = 1000000
DIM = 64
B = 16384
C = 20
K = 50

NC = 2
NS = 16
NW = NC * NS
EPW = B // NW
CHUNK = 16
NCHUNK = EPW // CHUNK
XROWS = CHUNK * C
NROWS = CHUNK * K
XCOLS = 40
NCOLS = 100
RTILE = 10
DBLK = 16


def _scores(iota16, rows_v, crow_v, outf_v, rows_per_elem, negate):
    iota_rpe = iota16 * rows_per_elem
    row_idx_base = iota16 * (C if rows_per_elem == C else K)
    zero = jnp.zeros((16,), jnp.float32)
    dvecs = [jnp.full((16,), d, jnp.int32) for d in range(DIM)]

    def rtile_body(rt, carry):
        r0 = rt * RTILE
        accs = [zero] * RTILE
        for blk in range(DIM // DBLK):
            cvals = [plsc.load_gather(crow_v, [iota16, dvecs[blk * DBLK + j]])
                     for j in range(DBLK)]
            if negate:
                cvals = [-cv for cv in cvals]
            for rr in range(RTILE):
                rowvec = iota_rpe + (r0 + rr)
                a = accs[rr]
                for j in range(DBLK):
                    x = plsc.load_gather(rows_v, [rowvec, dvecs[blk * DBLK + j]])
                    a = a + x * cvals[j]
                accs[rr] = a
        for rr in range(RTILE):
            idxvec = row_idx_base + (r0 + rr)
            plsc.store_scatter(outf_v, [idxvec], accs[rr])
        return carry

    lax.fori_loop(0, rows_per_elem // RTILE, rtile_body, 0)


def _sc_body(cw_hbm, ctx_hbm, neg_hbm, ctab_hbm, xtab_hbm,
             pos_hbm, nsc_hbm,
             cidx_v, xidx_v, nidx_v, crow_v, xrow_v, nrow_v,
             posf_v, negf_v, sem):
    wid = lax.axis_index("s") * NC + lax.axis_index("c")
    iota16 = lax.iota(jnp.int32, 16)

    def chunk_body(t, carry):
        base = pl.multiple_of(wid * EPW + t * CHUNK, CHUNK)
        pltpu.sync_copy(cw_hbm.at[pl.ds(base, CHUNK)], cidx_v)
        pltpu.sync_copy(
            ctx_hbm.at[pl.ds(pl.multiple_of(base * C // XCOLS, 8),
                             XROWS // XCOLS)], xidx_v)
        pltpu.sync_copy(
            neg_hbm.at[pl.ds(pl.multiple_of(base * K // NCOLS, 8),
                             NROWS // NCOLS)], nidx_v)
        dmas = [pltpu.async_copy(ctab_hbm.at[cidx_v], crow_v, sem)]
        for j in range(XROWS // XCOLS):
            dmas.append(pltpu.async_copy(
                xtab_hbm.at[xidx_v.at[j]],
                xrow_v.at[pl.ds(j * XCOLS, XCOLS)], sem))
        for j in range(NROWS // NCOLS):
            dmas.append(pltpu.async_copy(
                xtab_hbm.at[nidx_v.at[j]],
                nrow_v.at[pl.ds(j * NCOLS, NCOLS)], sem))
        for d in dmas:
            d.wait()

        _scores(iota16, xrow_v, crow_v, posf_v, C, negate=False)
        _scores(iota16, nrow_v, crow_v, negf_v, K, negate=True)

        pltpu.sync_copy(posf_v, pos_hbm.at[pl.ds(base * C, XROWS)])
        pltpu.sync_copy(negf_v, nsc_hbm.at[pl.ds(base * K, NROWS)])
        return carry

    lax.fori_loop(0, NCHUNK, chunk_body, 0)


_sc_kernel = pl.kernel(
    _sc_body,
    out_type=(
        jax.ShapeDtypeStruct((B * C,), jnp.float32),
        jax.ShapeDtypeStruct((B * K,), jnp.float32),
    ),
    mesh=plsc.VectorSubcoreMesh(core_axis_name="c", subcore_axis_name="s"),
    compiler_params=pltpu.CompilerParams(use_tc_tiling_on_sc=False,
                                         needs_layout_passes=False),
    scratch_types=[
        pltpu.VMEM((CHUNK,), jnp.int32),
        pltpu.VMEM((XROWS // XCOLS, XCOLS), jnp.int32),
        pltpu.VMEM((NROWS // NCOLS, NCOLS), jnp.int32),
        pltpu.VMEM((CHUNK, DIM), jnp.float32),
        pltpu.VMEM((XROWS, DIM), jnp.float32),
        pltpu.VMEM((NROWS, DIM), jnp.float32),
        pltpu.VMEM((XROWS,), jnp.float32),
        pltpu.VMEM((NROWS,), jnp.float32),
        pltpu.SemaphoreType.DMA,
    ],
)


@jax.jit
def kernel(center_word, context_words, negative_words, centerword_table,
           contextword_table):
    cw = center_word.astype(jnp.int32)
    ctx = context_words.astype(jnp.int32).reshape(B * C // XCOLS, XCOLS)
    neg = negative_words.astype(jnp.int32).reshape(B * K // NCOLS, NCOLS)
    pos, nsc = _sc_kernel(cw, ctx, neg, centerword_table, contextword_table)
    return pos.reshape(B, C), nsc.reshape(B, K)
